# baseline (speedup 1.0000x reference)
import functools

import jax
import jax.numpy as jnp
from jax import lax
from jax.experimental import pallas as pl
from jax.experimental.pallas import tpu as pltpu
from jax.experimental.pallas import tpu_sc as plsc

NC = 2
NS = 16
NW = NC * NS
CHUNK = 128


@jax.jit
def _run(x_flat, W):
    N = x_flat.shape[0]
    V, D = W.shape
    n_per_w = N // NW
    n_chunks = n_per_w // CHUNK
    x3 = x_flat.reshape(NW, n_chunks, CHUNK)

    mesh = plsc.VectorSubcoreMesh(core_axis_name="c", subcore_axis_name="s")

    @functools.partial(
        pl.kernel,
        out_type=jax.ShapeDtypeStruct((N, D), jnp.float32),
        mesh=mesh,
        scratch_types=[
            pltpu.VMEM((n_chunks, CHUNK), jnp.int32),
            pltpu.VMEM((CHUNK, D), jnp.float32),
            pltpu.VMEM((CHUNK, D), jnp.float32),
            pltpu.SemaphoreType.DMA,
            pltpu.SemaphoreType.DMA,
            pltpu.SemaphoreType.DMA,
            pltpu.SemaphoreType.DMA,
        ],
    )
    def k(x_hbm, w_hbm, out_hbm, idx_v, buf0, buf1, g0, g1, w0, w1):
        cid = lax.axis_index("c")
        sid = lax.axis_index("s")
        wid = cid * NS + sid
        base = wid * n_per_w

        pltpu.sync_copy(x_hbm.at[wid], idx_v)

        bufs = (buf0, buf1)
        gsems = (g0, g1)
        wsems = (w0, w1)

        pltpu.async_copy(w_hbm.at[idx_v.at[0]], buf0, g0)
        pltpu.async_copy(w_hbm.at[idx_v.at[1]], buf1, g1)

        def step(i, _):
            for b in range(2):
                j = 2 * i + b
                pltpu.make_async_copy(w_hbm.at[idx_v.at[0]], bufs[b],
                                      gsems[b]).wait()
                pltpu.async_copy(
                    bufs[b], out_hbm.at[pl.ds(base + j * CHUNK, CHUNK)],
                    wsems[b])
                jn = j + 2

                @pl.when(jn < n_chunks)
                def _():
                    pltpu.make_async_copy(
                        bufs[b], out_hbm.at[pl.ds(base, CHUNK)],
                        wsems[b]).wait()
                    pltpu.async_copy(w_hbm.at[idx_v.at[jn]], bufs[b],
                                     gsems[b])
            return 0

        lax.fori_loop(0, n_chunks // 2, step, 0)
        pltpu.make_async_copy(buf0, out_hbm.at[pl.ds(base, CHUNK)], w0).wait()
        pltpu.make_async_copy(buf1, out_hbm.at[pl.ds(base, CHUNK)], w1).wait()

    return k(x3, W)


def kernel(x, W):
    x = x.astype(jnp.int32)
    B, S = x.shape
    D = W.shape[1]
    out = _run(x.reshape(B * S), W)
    return out.reshape(B, S, D)
